# Initial kernel scaffold; baseline (speedup 1.0000x reference)
import jax
import jax.numpy as jnp
from jax.experimental import pallas as pl


def kernel(cell_q, edge_intra, edge_inter, params):
    raise NotImplementedError("write your pallas kernel here")


# speedup vs baseline: 2.0165x; 2.0165x over previous
import functools

import jax
import jax.numpy as jnp
from jax import lax
from jax.experimental import pallas as pl
from jax.experimental.pallas import tpu as pltpu
from jax.experimental.pallas import tpu_sc as plsc

N = 10000
H = 128
E = 160000
STEPS = 4

NW = 32
EW = 2 * E // NW
KG = 80
ES = E // 16
KS = 80
NSR = 624
NTAIL = N - 16 * NSR
BN = 1000
BE = 2000

_SC_MESH = dict(core_axis_name="c", subcore_axis_name="s")


def _sc_gather_add(tab_a, tab_b, src, dst):
    @functools.partial(
        pl.kernel,
        mesh=plsc.VectorSubcoreMesh(**_SC_MESH),
        out_type=jax.ShapeDtypeStruct((2 * E, H), jnp.float32),
        scratch_types=[
            pltpu.VMEM((KG,), jnp.int32),
            pltpu.VMEM((KG,), jnp.int32),
            pltpu.VMEM((KG, H), jnp.float32),
            pltpu.VMEM((KG, H), jnp.float32),
            pltpu.SemaphoreType.DMA,
            pltpu.SemaphoreType.DMA,
        ],
    )
    def k(ta, tb, s_idx, d_idx, z_out, si, di, ba, bb, sa, sb):
        wid = lax.axis_index("s") * 2 + lax.axis_index("c")
        base = wid * EW

        def chunk(i, carry):
            off = pl.multiple_of(base + i * KG, 8)
            pltpu.sync_copy(s_idx.at[pl.ds(off, KG)], si)
            pltpu.sync_copy(d_idx.at[pl.ds(off, KG)], di)
            cpa = pltpu.async_copy(ta.at[si], ba, sa)
            cpb = pltpu.async_copy(tb.at[di], bb, sb)
            cpa.wait()
            cpb.wait()

            def addrow(r, c2):
                for c in range(H // 16):
                    plsc.addupdate(ba.at[r, pl.ds(c * 16, 16)],
                                   bb[r, pl.ds(c * 16, 16)])
                return c2

            lax.fori_loop(0, KG, addrow, 0)
            pltpu.sync_copy(ba, z_out.at[pl.ds(off, KG)])
            return carry

        lax.fori_loop(0, EW // KG, chunk, 0)

    return k(tab_a, tab_b, src, dst)


def _sc_scatter_add(m_all, dst2, zeros_nh):
    @functools.partial(
        pl.kernel,
        mesh=plsc.VectorSubcoreMesh(**_SC_MESH),
        out_type=jax.ShapeDtypeStruct((2, N, H), jnp.float32),
        scratch_types=[
            pltpu.VMEM((KS,), jnp.int32),
            pltpu.VMEM((KS, H), jnp.float32),
            pltpu.VMEM_SHARED((N, H), jnp.float32),
        ],
    )
    def k(m_hbm, d_idx, z_hbm, agg_out, iv, buf, agg_sh):
        c = lax.axis_index("c")
        s = lax.axis_index("s")
        r0 = pl.multiple_of(s * NSR, 8)
        pltpu.sync_copy(z_hbm.at[pl.ds(r0, NSR)], agg_sh.at[pl.ds(r0, NSR)])

        @pl.when(s == 15)
        def _zero_tail():
            pltpu.sync_copy(z_hbm.at[pl.ds(16 * NSR, NTAIL)],
                            agg_sh.at[pl.ds(16 * NSR, NTAIL)])

        plsc.subcore_barrier()

        def chunk(i, carry):
            off = pl.multiple_of(s * ES + i * KS, 8)
            ioff = pl.multiple_of(c * E + off, 8)
            pltpu.sync_copy(d_idx.at[pl.ds(ioff, KS)], iv)
            pltpu.sync_copy(m_hbm.at[c, pl.ds(off, KS)], buf)
            pltpu.sync_copy(buf, agg_sh.at[iv], add=True)
            return carry

        lax.fori_loop(0, ES // KS, chunk, 0)
        plsc.subcore_barrier()
        pltpu.sync_copy(agg_sh.at[pl.ds(r0, NSR)],
                        agg_out.at[c, pl.ds(r0, NSR)])

        @pl.when(s == 15)
        def _write_tail():
            pltpu.sync_copy(agg_sh.at[pl.ds(16 * NSR, NTAIL)],
                            agg_out.at[c, pl.ds(16 * NSR, NTAIL)])

    return k(m_all, dst2, zeros_nh)


def _tc_init(cq2, emb, w1s, w1d, b1):
    def body(q_ref, e_ref, ws_ref, wd_ref, b1_ref, x_ref, a_ref, bt_ref):
        q = q_ref[...]
        e = e_ref[...]
        x = jnp.where(q == 0, e[0:1, :], jnp.where(q == 1, e[1:2, :], e[2:3, :]))
        x_ref[...] = x
        for t in range(2):
            a_ref[t] = jnp.dot(x, ws_ref[t], preferred_element_type=jnp.float32)
            bt_ref[t] = (jnp.dot(x, wd_ref[t], preferred_element_type=jnp.float32)
                         + b1_ref[t])

    return pl.pallas_call(
        body,
        grid=(N // BN,),
        in_specs=[
            pl.BlockSpec((BN, 1), lambda i: (i, 0)),
            pl.BlockSpec((3, H), lambda i: (0, 0)),
            pl.BlockSpec((2, H, H), lambda i: (0, 0, 0)),
            pl.BlockSpec((2, H, H), lambda i: (0, 0, 0)),
            pl.BlockSpec((2, 1, H), lambda i: (0, 0, 0)),
        ],
        out_specs=[
            pl.BlockSpec((BN, H), lambda i: (i, 0)),
            pl.BlockSpec((2, BN, H), lambda i: (0, i, 0)),
            pl.BlockSpec((2, BN, H), lambda i: (0, i, 0)),
        ],
        out_shape=[
            jax.ShapeDtypeStruct((N, H), jnp.float32),
            jax.ShapeDtypeStruct((2, N, H), jnp.float32),
            jax.ShapeDtypeStruct((2, N, H), jnp.float32),
        ],
    )(cq2, emb, w1s, w1d, b1)


def _tc_mlp(z_all, w2, b2, w3, b3, w4, b4):
    def body(z_ref, w2r, b2r, w3r, b3r, w4r, b4r, m_ref):
        t = jnp.maximum(z_ref[0], 0.0)
        t = jnp.maximum(
            jnp.dot(t, w2r[0], preferred_element_type=jnp.float32) + b2r[0], 0.0)
        t = jnp.maximum(
            jnp.dot(t, w3r[0], preferred_element_type=jnp.float32) + b3r[0], 0.0)
        m_ref[0] = jnp.dot(t, w4r[0], preferred_element_type=jnp.float32) + b4r[0]

    wspec = pl.BlockSpec((1, H, H), lambda t, i: (t, 0, 0))
    bspec = pl.BlockSpec((1, 1, H), lambda t, i: (t, 0, 0))
    return pl.pallas_call(
        body,
        grid=(2, E // BE),
        in_specs=[pl.BlockSpec((1, BE, H), lambda t, i: (t, i, 0)),
                  wspec, bspec, wspec, bspec, wspec, bspec],
        out_specs=pl.BlockSpec((1, BE, H), lambda t, i: (t, i, 0)),
        out_shape=jax.ShapeDtypeStruct((2, E, H), jnp.float32),
    )(z_all, w2, b2, w3, b3, w4, b4)


def _tc_lstm(x, agg, rh, rc, wih, whh, w1s, w1d, b1, wsc):
    def body(x_ref, g_ref, h_ref, c_ref, wih_ref, whh_ref, ws_ref, wd_ref,
             b1_ref, sc_ref, h2_ref, c2_ref, a_ref, bt_ref, lg_ref):
        xb = x_ref[...]
        gates = (
            jnp.dot(xb, wih_ref[0:H], preferred_element_type=jnp.float32)
            + jnp.dot(g_ref[0], wih_ref[H:2 * H],
                      preferred_element_type=jnp.float32)
            + jnp.dot(g_ref[1], wih_ref[2 * H:3 * H],
                      preferred_element_type=jnp.float32)
            + jnp.dot(h_ref[...], whh_ref[...],
                      preferred_element_type=jnp.float32))
        i_g = gates[:, 0:H]
        f_g = gates[:, H:2 * H]
        g_g = gates[:, 2 * H:3 * H]
        o_g = gates[:, 3 * H:4 * H]
        c_new = (jax.nn.sigmoid(f_g) * c_ref[...]
                 + jax.nn.sigmoid(i_g) * jnp.tanh(g_g))
        h_new = jax.nn.sigmoid(o_g) * jnp.tanh(c_new)
        c2_ref[...] = c_new
        h2_ref[...] = h_new
        for t in range(2):
            a_ref[t] = jnp.dot(h_new, ws_ref[t],
                               preferred_element_type=jnp.float32)
            bt_ref[t] = (jnp.dot(h_new, wd_ref[t],
                                 preferred_element_type=jnp.float32)
                         + b1_ref[t])
        lg_ref[...] = jnp.sum(h_new * sc_ref[...], axis=1, keepdims=True)

    return pl.pallas_call(
        body,
        grid=(N // BN,),
        in_specs=[
            pl.BlockSpec((BN, H), lambda i: (i, 0)),
            pl.BlockSpec((2, BN, H), lambda i: (0, i, 0)),
            pl.BlockSpec((BN, H), lambda i: (i, 0)),
            pl.BlockSpec((BN, H), lambda i: (i, 0)),
            pl.BlockSpec((3 * H, 4 * H), lambda i: (0, 0)),
            pl.BlockSpec((H, 4 * H), lambda i: (0, 0)),
            pl.BlockSpec((2, H, H), lambda i: (0, 0, 0)),
            pl.BlockSpec((2, H, H), lambda i: (0, 0, 0)),
            pl.BlockSpec((2, 1, H), lambda i: (0, 0, 0)),
            pl.BlockSpec((1, H), lambda i: (0, 0)),
        ],
        out_specs=[
            pl.BlockSpec((BN, H), lambda i: (i, 0)),
            pl.BlockSpec((BN, H), lambda i: (i, 0)),
            pl.BlockSpec((2, BN, H), lambda i: (0, i, 0)),
            pl.BlockSpec((2, BN, H), lambda i: (0, i, 0)),
            pl.BlockSpec((BN, 1), lambda i: (i, 0)),
        ],
        out_shape=[
            jax.ShapeDtypeStruct((N, H), jnp.float32),
            jax.ShapeDtypeStruct((N, H), jnp.float32),
            jax.ShapeDtypeStruct((2, N, H), jnp.float32),
            jax.ShapeDtypeStruct((2, N, H), jnp.float32),
            jax.ShapeDtypeStruct((N, 1), jnp.float32),
        ],
    )(x, agg, rh, rc, wih, whh, w1s, w1d, b1, wsc)


def kernel(cell_q, edge_intra, edge_inter, params):
    p = params
    cq2 = cell_q.astype(jnp.int32).reshape(N, 1)
    ei = edge_intra.astype(jnp.int32)
    ee = edge_inter.astype(jnp.int32)
    src = jnp.concatenate([ei[0], ee[0] + N])
    dst_g = jnp.concatenate([ei[1], ee[1] + N])
    dst_s = jnp.concatenate([ei[1], ee[1]])

    w1s = jnp.stack([p['intra_Ws'][0][:H], p['inter_Ws'][0][:H]])
    w1d = jnp.stack([p['intra_Ws'][0][H:], p['inter_Ws'][0][H:]])
    b1 = jnp.stack([p['intra_bs'][0], p['inter_bs'][0]])[:, None, :]
    w2 = jnp.stack([p['intra_Ws'][1], p['inter_Ws'][1]])
    b2 = jnp.stack([p['intra_bs'][1], p['inter_bs'][1]])[:, None, :]
    w3 = jnp.stack([p['intra_Ws'][2], p['inter_Ws'][2]])
    b3 = jnp.stack([p['intra_bs'][2], p['inter_bs'][2]])[:, None, :]
    w4 = jnp.stack([p['intra_Ws'][3], p['inter_Ws'][3]])
    b4 = jnp.stack([p['intra_bs'][3], p['inter_bs'][3]])[:, None, :]
    zeros_nh = jnp.zeros((N, H), jnp.float32)

    x, A, B = _tc_init(cq2, p['digit_embed'], w1s, w1d, b1)
    rh = zeros_nh
    rc = zeros_nh
    lg = None
    for _ in range(STEPS):
        z = _sc_gather_add(A.reshape(2 * N, H), B.reshape(2 * N, H),
                           src, dst_g).reshape(2, E, H)
        m = _tc_mlp(z, w2, b2, w3, b3, w4, b4)
        agg = _sc_scatter_add(m, dst_s, zeros_nh)
        rh, rc, A, B, lg = _tc_lstm(x, agg, rh, rc, p['W_ih'], p['W_hh'],
                                    w1s, w1d, b1, p['w_score'][None, :])
    return lg[:, 0]
